# initial kernel scaffold (unmeasured)
import jax
import jax.numpy as jnp
from jax import lax
from jax.experimental import pallas as pl
from jax.experimental.pallas import tpu as pltpu


def kernel(
    x,
):
    def body(*refs):
        pass

    out_shape = jax.ShapeDtypeStruct(..., jnp.float32)
    return pl.pallas_call(body, out_shape=out_shape)(...)


# baseline (device time: 57771 ns/iter reference)
import jax
import jax.numpy as jnp
from jax import lax
from jax.experimental import pallas as pl
from jax.experimental.pallas import tpu as pltpu


def kernel(x):
    m, n = x.shape

    def body(x_ref, out_ref, comm_ref, send_sems, recv_sems):
        my_x = lax.axis_index("x")
        my_y = lax.axis_index("y")

        rdma1 = pltpu.make_async_remote_copy(
            src_ref=x_ref,
            dst_ref=comm_ref.at[0],
            send_sem=send_sems.at[0],
            recv_sem=recv_sems.at[0],
            device_id=(1 - my_x, my_y),
            device_id_type=pl.DeviceIdType.MESH,
        )
        rdma1.start()
        rdma1.wait()
        out_ref[...] = x_ref[...] + comm_ref[0]

        rdma2 = pltpu.make_async_remote_copy(
            src_ref=out_ref,
            dst_ref=comm_ref.at[1],
            send_sem=send_sems.at[1],
            recv_sem=recv_sems.at[1],
            device_id=(my_x, 1 - my_y),
            device_id_type=pl.DeviceIdType.MESH,
        )
        rdma2.start()
        rdma2.wait()
        out_ref[...] = out_ref[...] + comm_ref[1]

    return pl.pallas_call(
        body,
        out_shape=jax.ShapeDtypeStruct((m, n), x.dtype),
        in_specs=[pl.BlockSpec(memory_space=pltpu.VMEM)],
        out_specs=pl.BlockSpec(memory_space=pltpu.VMEM),
        scratch_shapes=[
            pltpu.VMEM((2, m, n), x.dtype),
            pltpu.SemaphoreType.DMA((2,)),
            pltpu.SemaphoreType.DMA((2,)),
        ],
    )(x)


# device time: 30787 ns/iter; 1.8765x vs baseline; 1.8765x over previous
import jax
import jax.numpy as jnp
from jax import lax
from jax.experimental import pallas as pl
from jax.experimental.pallas import tpu as pltpu


def kernel(x):
    m, n = x.shape
    mh = m // 2
    q = m // 4

    def body(x_ref, out_ref, comm_ref, send_sems, recv_sems):
        my_x = lax.axis_index("x")
        my_y = lax.axis_index("y")
        x_nbr = (1 - my_x, my_y)
        y_nbr = (my_x, 1 - my_y)

        rA = pl.ds(q * my_x, q)
        rA_o = pl.ds(q * (1 - my_x), q)
        rB = pl.ds(mh + q * my_y, q)
        rB_o = pl.ds(mh + q * (1 - my_y), q)

        def rdma(src, dst, i, dev):
            return pltpu.make_async_remote_copy(
                src_ref=src,
                dst_ref=dst,
                send_sem=send_sems.at[i],
                recv_sem=recv_sems.at[i],
                device_id=dev,
                device_id_type=pl.DeviceIdType.MESH,
            )

        a1 = rdma(x_ref.at[rA_o], comm_ref.at[0], 0, x_nbr)
        b1 = rdma(x_ref.at[rB_o], comm_ref.at[1], 1, y_nbr)
        a1.start()
        b1.start()
        a1.wait()
        b1.wait()
        out_ref[rA, :] = x_ref[rA, :] + comm_ref[0]
        out_ref[rB, :] = x_ref[rB, :] + comm_ref[1]

        a2 = rdma(out_ref.at[rA], comm_ref.at[2], 2, y_nbr)
        b2 = rdma(out_ref.at[rB], comm_ref.at[3], 3, x_nbr)
        a2.start()
        b2.start()
        a2.wait()
        b2.wait()
        out_ref[rA, :] = out_ref[rA, :] + comm_ref[2]
        out_ref[rB, :] = out_ref[rB, :] + comm_ref[3]

        a3 = rdma(out_ref.at[rA], out_ref.at[rA], 4, x_nbr)
        b3 = rdma(out_ref.at[rB], out_ref.at[rB], 5, y_nbr)
        a3.start()
        b3.start()
        a3.wait()
        b3.wait()

    return pl.pallas_call(
        body,
        out_shape=jax.ShapeDtypeStruct((m, n), x.dtype),
        in_specs=[pl.BlockSpec(memory_space=pltpu.VMEM)],
        out_specs=pl.BlockSpec(memory_space=pltpu.VMEM),
        scratch_shapes=[
            pltpu.VMEM((4, q, n), x.dtype),
            pltpu.SemaphoreType.DMA((6,)),
            pltpu.SemaphoreType.DMA((6,)),
        ],
    )(x)


# device time: 26939 ns/iter; 2.1445x vs baseline; 1.1428x over previous
import jax
import jax.numpy as jnp
from jax import lax
from jax.experimental import pallas as pl
from jax.experimental.pallas import tpu as pltpu


def kernel(x):
    m, n = x.shape
    mh = m // 2
    q = m // 4

    def body(x_ref, out_ref, comm_ref, send_sems, recv_sems):
        my_x = lax.axis_index("x")
        my_y = lax.axis_index("y")
        x_nbr = (1 - my_x, my_y)
        y_nbr = (my_x, 1 - my_y)

        rA = pl.ds(q * my_x, q)
        rA_o = pl.ds(q * (1 - my_x), q)
        rB = pl.ds(mh + q * my_y, q)
        rB_o = pl.ds(mh + q * (1 - my_y), q)

        def rdma(src, dst, i, dev):
            return pltpu.make_async_remote_copy(
                src_ref=src,
                dst_ref=dst,
                send_sem=send_sems.at[i],
                recv_sem=recv_sems.at[i],
                device_id=dev,
                device_id_type=pl.DeviceIdType.MESH,
            )

        barrier_sem = pltpu.get_barrier_semaphore()
        pl.semaphore_signal(
            barrier_sem, inc=1, device_id=x_nbr,
            device_id_type=pl.DeviceIdType.MESH,
        )
        pl.semaphore_signal(
            barrier_sem, inc=1, device_id=y_nbr,
            device_id_type=pl.DeviceIdType.MESH,
        )
        pl.semaphore_wait(barrier_sem, 2)

        a1 = rdma(x_ref.at[rA_o], comm_ref.at[0], 0, x_nbr)
        b1 = rdma(x_ref.at[rB_o], comm_ref.at[1], 1, y_nbr)
        a1.start()
        b1.start()

        a1.wait_recv()
        out_ref[rA, :] = x_ref[rA, :] + comm_ref[0]
        a2 = rdma(out_ref.at[rA], comm_ref.at[2], 2, y_nbr)
        a2.start()

        b1.wait_recv()
        out_ref[rB, :] = x_ref[rB, :] + comm_ref[1]
        b2 = rdma(out_ref.at[rB], comm_ref.at[3], 3, x_nbr)
        b2.start()

        a2.wait_recv()
        out_ref[rA, :] = out_ref[rA, :] + comm_ref[2]
        a3 = rdma(out_ref.at[rA], out_ref.at[rA], 4, x_nbr)
        a3.start()

        b2.wait_recv()
        out_ref[rB, :] = out_ref[rB, :] + comm_ref[3]
        b3 = rdma(out_ref.at[rB], out_ref.at[rB], 5, y_nbr)
        b3.start()

        a3.wait_recv()
        b3.wait_recv()

        a1.wait_send()
        b1.wait_send()
        a2.wait_send()
        b2.wait_send()
        a3.wait_send()
        b3.wait_send()

    return pl.pallas_call(
        body,
        out_shape=jax.ShapeDtypeStruct((m, n), x.dtype),
        in_specs=[pl.BlockSpec(memory_space=pltpu.VMEM)],
        out_specs=pl.BlockSpec(memory_space=pltpu.VMEM),
        scratch_shapes=[
            pltpu.VMEM((4, q, n), x.dtype),
            pltpu.SemaphoreType.DMA((6,)),
            pltpu.SemaphoreType.DMA((6,)),
        ],
        compiler_params=pltpu.CompilerParams(collective_id=0),
    )(x)


# device time: 24787 ns/iter; 2.3307x vs baseline; 1.0868x over previous
import jax
import jax.numpy as jnp
from jax import lax
from jax.experimental import pallas as pl
from jax.experimental.pallas import tpu as pltpu

NUM_CHUNKS = 2


def kernel(x):
    m, n = x.shape
    mh = m // 2
    q = m // 4
    C = NUM_CHUNKS
    qc = q // C

    def body(x_ref, out_ref, comm_ref, send_sems, recv_sems):
        my_x = lax.axis_index("x")
        my_y = lax.axis_index("y")
        x_nbr = (1 - my_x, my_y)
        y_nbr = (my_x, 1 - my_y)

        sA = q * my_x
        sA_o = q * (1 - my_x)
        sB = mh + q * my_y
        sB_o = mh + q * (1 - my_y)

        def rows(start, c):
            return pl.ds(start + c * qc, qc)

        def rdma(src, dst, k, c, dev):
            return pltpu.make_async_remote_copy(
                src_ref=src,
                dst_ref=dst,
                send_sem=send_sems.at[k, c],
                recv_sem=recv_sems.at[k, c],
                device_id=dev,
                device_id_type=pl.DeviceIdType.MESH,
            )

        barrier_sem = pltpu.get_barrier_semaphore()
        pl.semaphore_signal(
            barrier_sem, inc=1, device_id=x_nbr,
            device_id_type=pl.DeviceIdType.MESH,
        )
        pl.semaphore_signal(
            barrier_sem, inc=1, device_id=y_nbr,
            device_id_type=pl.DeviceIdType.MESH,
        )
        pl.semaphore_wait(barrier_sem, 2)

        a1 = [rdma(x_ref.at[rows(sA_o, c)], comm_ref.at[0, c], 0, c, x_nbr)
              for c in range(C)]
        b1 = [rdma(x_ref.at[rows(sB_o, c)], comm_ref.at[1, c], 1, c, y_nbr)
              for c in range(C)]
        for op in a1 + b1:
            op.start()

        a2 = []
        b2 = []
        for c in range(C):
            a1[c].wait_recv()
            out_ref[rows(sA, c), :] = x_ref[rows(sA, c), :] + comm_ref[0, c]
            op = rdma(out_ref.at[rows(sA, c)], comm_ref.at[2, c], 2, c, y_nbr)
            op.start()
            a2.append(op)

            b1[c].wait_recv()
            out_ref[rows(sB, c), :] = x_ref[rows(sB, c), :] + comm_ref[1, c]
            op = rdma(out_ref.at[rows(sB, c)], comm_ref.at[3, c], 3, c, x_nbr)
            op.start()
            b2.append(op)

        a3 = []
        b3 = []
        for c in range(C):
            a2[c].wait_recv()
            out_ref[rows(sA, c), :] = out_ref[rows(sA, c), :] + comm_ref[2, c]
            op = rdma(out_ref.at[rows(sA, c)], out_ref.at[rows(sA, c)],
                      4, c, x_nbr)
            op.start()
            a3.append(op)

            b2[c].wait_recv()
            out_ref[rows(sB, c), :] = out_ref[rows(sB, c), :] + comm_ref[3, c]
            op = rdma(out_ref.at[rows(sB, c)], out_ref.at[rows(sB, c)],
                      5, c, y_nbr)
            op.start()
            b3.append(op)

        for c in range(C):
            a3[c].wait_recv()
            b3[c].wait_recv()

        for op in a1 + b1 + a2 + b2 + a3 + b3:
            op.wait_send()

    return pl.pallas_call(
        body,
        out_shape=jax.ShapeDtypeStruct((m, n), x.dtype),
        in_specs=[pl.BlockSpec(memory_space=pltpu.VMEM)],
        out_specs=pl.BlockSpec(memory_space=pltpu.VMEM),
        scratch_shapes=[
            pltpu.VMEM((4, C, qc, n), x.dtype),
            pltpu.SemaphoreType.DMA((6, C)),
            pltpu.SemaphoreType.DMA((6, C)),
        ],
        compiler_params=pltpu.CompilerParams(collective_id=0),
    )(x)
